# baseline (speedup 1.0000x reference)
import jax
import jax.numpy as jnp
from jax.experimental import pallas as pl
from jax.experimental.pallas import tpu as pltpu

_NPOINT = 1024
_RADIUS = 0.2
_NSAMPLE = 32


def _fps_body(x_ref, y_ref, z_ref, idx_ref, cx_ref, cy_ref, cz_ref,
              dist_ref):
    B, N = x_ref.shape
    colid = jax.lax.broadcasted_iota(jnp.int32, (B, N), 1)
    x = x_ref[:]
    y = y_ref[:]
    z = z_ref[:]
    dist_ref[:] = jnp.full((B, N), 1e10, jnp.float32)

    def body(t, far):
        onehot = colid == far
        cx = jnp.sum(jnp.where(onehot, x, 0.0), axis=1, keepdims=True)
        cy = jnp.sum(jnp.where(onehot, y, 0.0), axis=1, keepdims=True)
        cz = jnp.sum(jnp.where(onehot, z, 0.0), axis=1, keepdims=True)
        dx = x - cx
        dy = y - cy
        dz = z - cz
        d = (dx * dx + dz * dz) + dy * dy
        dist = jnp.minimum(dist_ref[:], d)
        dist_ref[:] = dist
        m = jnp.max(dist, axis=1, keepdims=True)
        nxt = jnp.min(jnp.where(dist == m, colid, N), axis=1)
        idx_ref[pl.ds(t, 1), :] = far[:, 0].reshape(1, B)
        cx_ref[pl.ds(t, 1), :] = cx[:, 0].reshape(1, B)
        cy_ref[pl.ds(t, 1), :] = cy[:, 0].reshape(1, B)
        cz_ref[pl.ds(t, 1), :] = cz[:, 0].reshape(1, B)
        return nxt[:, None].astype(jnp.int32)

    jax.lax.fori_loop(0, _NPOINT, body, jnp.zeros((B, 1), jnp.int32))


def _fps_pallas(xyz):
    B, N, _ = xyz.shape
    x = xyz[:, :, 0]
    y = xyz[:, :, 1]
    z = xyz[:, :, 2]
    out_shape = (
        jax.ShapeDtypeStruct((_NPOINT, B), jnp.int32),
        jax.ShapeDtypeStruct((_NPOINT, B), jnp.float32),
        jax.ShapeDtypeStruct((_NPOINT, B), jnp.float32),
        jax.ShapeDtypeStruct((_NPOINT, B), jnp.float32),
    )
    idx_t, cx_t, cy_t, cz_t = pl.pallas_call(
        _fps_body,
        out_shape=out_shape,
        scratch_shapes=[pltpu.VMEM((B, N), jnp.float32)],
    )(x, y, z)
    fps_idx = idx_t.T
    new_xyz = jnp.stack([cx_t.T, cy_t.T, cz_t.T], axis=-1)
    return fps_idx, new_xyz


def kernel(xyz, points):
    B, N, _ = xyz.shape
    fps_idx, new_xyz = _fps_pallas(xyz)

    a = new_xyz
    b = xyz
    a2 = jnp.sum(a * a, axis=-1)[:, :, None]
    b2 = jnp.sum(b * b, axis=-1)[:, None, :]
    bf = lambda v: jax.lax.optimization_barrier(
        v.astype(jnp.bfloat16)).astype(jnp.float32)
    ab = bf(a[:, :, None, 0]) * bf(b[:, None, :, 0]) + (
        bf(a[:, :, None, 1]) * bf(b[:, None, :, 1])
        + bf(a[:, :, None, 2]) * bf(b[:, None, :, 2])
    )
    sqd = a2 + b2 - 2.0 * ab
    mask = sqd < (_RADIUS * _RADIUS)
    ar = jnp.broadcast_to(jnp.arange(N, dtype=jnp.int32), mask.shape)
    cand = jnp.where(mask, ar, N)
    cand = jnp.sort(cand, axis=-1)[:, :, :_NSAMPLE]
    first = cand[:, :, :1]
    idx = jnp.where(cand == N, first, cand)
    grouped_xyz = jax.vmap(lambda p, i: p[i])(xyz, idx)
    grouped_xyz = grouped_xyz - new_xyz[:, :, None, :]
    grouped_points = jax.vmap(lambda p, i: p[i])(points, idx)
    new_points = jnp.concatenate([grouped_xyz, grouped_points], axis=-1)
    return new_xyz, new_points


# speedup vs baseline: 1.0003x; 1.0003x over previous
import jax
import jax.numpy as jnp
from jax.experimental import pallas as pl
from jax.experimental.pallas import tpu as pltpu

_NPOINT = 1024
_RADIUS = 0.2
_NSAMPLE = 32


def _fps_body(x_ref, y_ref, z_ref, idx_ref, cx_ref, cy_ref, cz_ref,
              dist_ref):
    B, N = x_ref.shape
    colid = jax.lax.broadcasted_iota(jnp.int32, (B, N), 1)
    x = x_ref[:]
    y = y_ref[:]
    z = z_ref[:]
    dist_ref[:] = jnp.full((B, N), 1e10, jnp.float32)

    def body(t, far):
        onehot = colid == far
        cx = jnp.sum(jnp.where(onehot, x, 0.0), axis=1, keepdims=True)
        cy = jnp.sum(jnp.where(onehot, y, 0.0), axis=1, keepdims=True)
        cz = jnp.sum(jnp.where(onehot, z, 0.0), axis=1, keepdims=True)
        dx = x - cx
        dy = y - cy
        dz = z - cz
        d = (dx * dx + dz * dz) + dy * dy
        dist = jnp.minimum(dist_ref[:], d)
        dist_ref[:] = dist
        m = jnp.max(dist, axis=1, keepdims=True)
        nxt = jnp.min(jnp.where(dist == m, colid, N), axis=1)
        idx_ref[pl.ds(t, 1), :] = far[:, 0].reshape(1, B)
        cx_ref[pl.ds(t, 1), :] = cx[:, 0].reshape(1, B)
        cy_ref[pl.ds(t, 1), :] = cy[:, 0].reshape(1, B)
        cz_ref[pl.ds(t, 1), :] = cz[:, 0].reshape(1, B)
        return nxt[:, None].astype(jnp.int32)

    jax.lax.fori_loop(0, _NPOINT, body, jnp.zeros((B, 1), jnp.int32))


def _fps_pallas(xyz):
    B, N, _ = xyz.shape
    x = xyz[:, :, 0]
    y = xyz[:, :, 1]
    z = xyz[:, :, 2]
    out_shape = (
        jax.ShapeDtypeStruct((_NPOINT, B), jnp.int32),
        jax.ShapeDtypeStruct((_NPOINT, B), jnp.float32),
        jax.ShapeDtypeStruct((_NPOINT, B), jnp.float32),
        jax.ShapeDtypeStruct((_NPOINT, B), jnp.float32),
    )
    idx_t, cx_t, cy_t, cz_t = pl.pallas_call(
        _fps_body,
        out_shape=out_shape,
        scratch_shapes=[pltpu.VMEM((B, N), jnp.float32)],
    )(x, y, z)
    fps_idx = idx_t.T
    new_xyz = jnp.stack([cx_t.T, cy_t.T, cz_t.T], axis=-1)
    return fps_idx, new_xyz


def kernel(xyz, points):
    B, N, _ = xyz.shape
    fps_idx, new_xyz = _fps_pallas(xyz)

    a = new_xyz
    b = xyz
    a2 = jnp.sum(a * a, axis=-1)[:, :, None]
    b2 = jnp.sum(b * b, axis=-1)[:, None, :]
    bf = lambda v: jax.lax.optimization_barrier(
        v.astype(jnp.bfloat16)).astype(jnp.float32)
    ab = bf(a[:, :, None, 0]) * bf(b[:, None, :, 0]) + (
        bf(a[:, :, None, 1]) * bf(b[:, None, :, 1])
        + bf(a[:, :, None, 2]) * bf(b[:, None, :, 2])
    )
    sqd = a2 + b2 - 2.0 * ab
    mask = sqd < (_RADIUS * _RADIUS)
    ar = jnp.broadcast_to(jnp.arange(N, dtype=jnp.int32), mask.shape)
    cand = jnp.where(mask, ar, N)
    neg_top, _ = jax.lax.top_k(-cand, _NSAMPLE)
    cand = -neg_top
    first = cand[:, :, :1]
    idx = jnp.where(cand == N, first, cand)
    grouped_xyz = jax.vmap(lambda p, i: p[i])(xyz, idx)
    grouped_xyz = grouped_xyz - new_xyz[:, :, None, :]
    grouped_points = jax.vmap(lambda p, i: p[i])(points, idx)
    new_points = jnp.concatenate([grouped_xyz, grouped_points], axis=-1)
    return new_xyz, new_points
